# baseline (speedup 1.0000x reference)
import functools

import jax
import jax.numpy as jnp
from jax import lax
from jax.experimental import pallas as pl
from jax.experimental.pallas import tpu as pltpu
from jax.experimental.pallas import tpu_sc as plsc

BATCH = 16384
MF_DIM = 64
MLP_DIM = 128

_NC = 2
_NS = 16
_NW = _NC * _NS
_BPW = BATCH // _NW
_CHUNK = 128
_NCHUNK = _BPW // _CHUNK


_MESH = plsc.VectorSubcoreMesh(
    core_axis_name="c", subcore_axis_name="s",
    num_cores=_NC, num_subcores=_NS)


def _sc_gather_mlp(user, item, mlp_u, mlp_i):
  @functools.partial(
      pl.kernel,
      out_type=[
          jax.ShapeDtypeStruct((BATCH, MLP_DIM), jnp.float32),
          jax.ShapeDtypeStruct((BATCH, MLP_DIM), jnp.float32),
      ],
      mesh=_MESH,
      scratch_types=[
          pltpu.VMEM((_BPW,), jnp.int32),
          pltpu.VMEM((_BPW,), jnp.int32),
          pltpu.VMEM((2, _CHUNK, MLP_DIM), jnp.float32),
          pltpu.VMEM((2, _CHUNK, MLP_DIM), jnp.float32),
          pltpu.SemaphoreType.DMA,
          pltpu.SemaphoreType.DMA,
      ],
  )
  def k(user_h, item_h, mlpu_h, mlpi_h, omlpu_h, omlpi_h,
        uidx, iidx, bufc, bufd, gsem, wsem):
    wid = lax.axis_index("s") * _NC + lax.axis_index("c")
    pltpu.sync_copy(user_h.at[pl.ds(wid * _BPW, _BPW)], uidx)
    pltpu.sync_copy(item_h.at[pl.ds(wid * _BPW, _BPW)], iidx)
    writes = [None, None]
    for c in range(_NCHUNK):
      b = c % 2
      base = wid * _BPW + c * _CHUNK
      if writes[b] is not None:
        for w in writes[b]:
          w.wait()
      uc = uidx.at[pl.ds(c * _CHUNK, _CHUNK)]
      ic = iidx.at[pl.ds(c * _CHUNK, _CHUNK)]
      cc = pltpu.async_copy(mlpu_h.at[uc], bufc.at[b], gsem)
      cd = pltpu.async_copy(mlpi_h.at[ic], bufd.at[b], gsem)
      cc.wait()
      wc = pltpu.async_copy(bufc.at[b], omlpu_h.at[pl.ds(base, _CHUNK)], wsem)
      cd.wait()
      wd = pltpu.async_copy(bufd.at[b], omlpi_h.at[pl.ds(base, _CHUNK)], wsem)
      writes[b] = (wc, wd)
    for ws in writes:
      for w in ws:
        w.wait()

  return k(user, item, mlp_u, mlp_i)


def _sc_gather_mf(user, item, mfcat):
  @functools.partial(
      pl.kernel,
      out_type=[
          jax.ShapeDtypeStruct((BATCH, 2 * MF_DIM), jnp.float32),
          jax.ShapeDtypeStruct((BATCH, 2 * MF_DIM), jnp.float32),
      ],
      mesh=_MESH,
      scratch_types=[
          pltpu.VMEM((_BPW,), jnp.int32),
          pltpu.VMEM((_BPW,), jnp.int32),
          pltpu.VMEM((2, _CHUNK, 2 * MF_DIM), jnp.float32),
          pltpu.VMEM((2, _CHUNK, 2 * MF_DIM), jnp.float32),
          pltpu.SemaphoreType.DMA,
          pltpu.SemaphoreType.DMA,
      ],
  )
  def k(user_h, item_h, mfcat_h, omfu_h, omfi_h,
        uidx, iidx, bufa, bufb, gsem, wsem):
    wid = lax.axis_index("s") * _NC + lax.axis_index("c")
    pltpu.sync_copy(user_h.at[pl.ds(wid * _BPW, _BPW)], uidx)
    pltpu.sync_copy(item_h.at[pl.ds(wid * _BPW, _BPW)], iidx)
    writes = [None, None]
    for c in range(_NCHUNK):
      b = c % 2
      base = wid * _BPW + c * _CHUNK
      if writes[b] is not None:
        for w in writes[b]:
          w.wait()
      uc = uidx.at[pl.ds(c * _CHUNK, _CHUNK)]
      ic = iidx.at[pl.ds(c * _CHUNK, _CHUNK)]
      ca = pltpu.async_copy(mfcat_h.at[uc], bufa.at[b], gsem)
      cb = pltpu.async_copy(mfcat_h.at[ic], bufb.at[b], gsem)
      ca.wait()
      wa = pltpu.async_copy(bufa.at[b], omfu_h.at[pl.ds(base, _CHUNK)], wsem)
      cb.wait()
      wb = pltpu.async_copy(bufb.at[b], omfi_h.at[pl.ds(base, _CHUNK)], wsem)
      writes[b] = (wa, wb)
    for ws in writes:
      for w in ws:
        w.wait()

  return k(user, item, mfcat)


_BT = 2048


_TBC = 512


def _tc_transpose_body(ut_ref, it_ref, out_ref):
  out_ref[...] = jnp.concatenate(
      [ut_ref[...].T, it_ref[...].T], axis=1)


def _tc_build_mfcat(mf_uT, mf_iT, n_rows):
  grid = (-(-n_rows // _TBC),)
  return pl.pallas_call(
      _tc_transpose_body,
      grid=grid,
      in_specs=[
          pl.BlockSpec((MF_DIM, _TBC), lambda i: (0, i)),
          pl.BlockSpec((MF_DIM, _TBC), lambda i: (0, i)),
      ],
      out_specs=pl.BlockSpec((_TBC, 2 * MF_DIM), lambda i: (i, 0)),
      out_shape=jax.ShapeDtypeStruct((n_rows, 2 * MF_DIM), jnp.float32),
      compiler_params=pltpu.CompilerParams(
          dimension_semantics=("arbitrary",)),
  )(mf_uT, mf_iT)


def _tc_body(mfu, mfi, mlpu, mlpi, w1u, w1i, b1, w2, b2, wo, bo, out):
  x = jnp.dot(mlpu[...], w1u[...], preferred_element_type=jnp.float32)
  x = x + jnp.dot(mlpi[...], w1i[...], preferred_element_type=jnp.float32)
  h1 = jnp.maximum(x + b1[...], 0.0)
  h2 = jnp.maximum(
      jnp.dot(h1, w2[...], preferred_element_type=jnp.float32) + b2[...], 0.0)
  g = mfu[...][:, :MF_DIM] * mfi[...][:, MF_DIM:]
  p = jnp.concatenate([g, h2], axis=1)
  z = jnp.sum(p * wo[...], axis=1, keepdims=True) + bo[...]
  out[...] = jax.nn.sigmoid(z)


def _tc_mlp(mfu, mfi, mlpu, mlpi, W1, b1, W2, b2, W_out, b_out):
  w1t = W1.T
  w1u = w1t[:MLP_DIM]
  w1i = w1t[MLP_DIM:]
  w2t = W2.T
  b1r = b1.reshape(1, -1)
  b2r = b2.reshape(1, -1)
  wo = W_out.reshape(1, -1)
  bo = b_out.reshape(1, 1)

  grid = (BATCH // _BT,)
  bspec_row = lambda d: pl.BlockSpec((_BT, d), lambda i: (i, 0))
  bspec_full = lambda s: pl.BlockSpec(s, lambda i: (0, 0))
  return pl.pallas_call(
      _tc_body,
      grid=grid,
      in_specs=[
          bspec_row(2 * MF_DIM), bspec_row(2 * MF_DIM),
          bspec_row(MLP_DIM), bspec_row(MLP_DIM),
          bspec_full((MLP_DIM, 128)), bspec_full((MLP_DIM, 128)),
          bspec_full((1, 128)),
          bspec_full((128, 64)), bspec_full((1, 64)),
          bspec_full((1, 128)), bspec_full((1, 1)),
      ],
      out_specs=pl.BlockSpec((_BT, 1), lambda i: (i, 0)),
      out_shape=jax.ShapeDtypeStruct((BATCH, 1), jnp.float32),
      compiler_params=pltpu.CompilerParams(
          dimension_semantics=("arbitrary",)),
  )(mfu, mfi, mlpu, mlpi, w1u, w1i, b1r, w2t, b2r, wo, bo)


def kernel(user, item, mf_emb_user, mf_emb_item, mlp_emb_user, mlp_emb_item,
           W1, b1, W2, b2, W_out, b_out):
  user = user.astype(jnp.int32)
  item = item.astype(jnp.int32)
  mfcat = _tc_build_mfcat(mf_emb_user.T, mf_emb_item.T,
                          mf_emb_user.shape[0])
  mlpu, mlpi = _sc_gather_mlp(user, item, mlp_emb_user, mlp_emb_item)
  mfu, mfi = _sc_gather_mf(user, item, mfcat)
  return _tc_mlp(mfu, mfi, mlpu, mlpi, W1, b1, W2, b2, W_out, b_out)


# speedup vs baseline: 1.2621x; 1.2621x over previous
import functools

import jax
import jax.numpy as jnp
from jax import lax
from jax.experimental import pallas as pl
from jax.experimental.pallas import tpu as pltpu
from jax.experimental.pallas import tpu_sc as plsc

BATCH = 16384
MF_DIM = 64
MLP_DIM = 128

_NC = 2
_NS = 16
_NW = _NC * _NS
_BPW = BATCH // _NW
_CHUNK = 128
_NCHUNK = _BPW // _CHUNK


_MESH = plsc.VectorSubcoreMesh(
    core_axis_name="c", subcore_axis_name="s",
    num_cores=_NC, num_subcores=_NS)


def _sc_gather_mlp(user, item, mlp_u, mlp_i):
  @functools.partial(
      pl.kernel,
      out_type=[
          jax.ShapeDtypeStruct((BATCH, MLP_DIM), jnp.float32),
          jax.ShapeDtypeStruct((BATCH, MLP_DIM), jnp.float32),
      ],
      mesh=_MESH,
      scratch_types=[
          pltpu.VMEM((_BPW,), jnp.int32),
          pltpu.VMEM((_BPW,), jnp.int32),
          pltpu.VMEM((2, _CHUNK, MLP_DIM), jnp.float32),
          pltpu.VMEM((2, _CHUNK, MLP_DIM), jnp.float32),
          pltpu.SemaphoreType.DMA,
          pltpu.SemaphoreType.DMA,
      ],
  )
  def k(user_h, item_h, mlpu_h, mlpi_h, omlpu_h, omlpi_h,
        uidx, iidx, bufc, bufd, gsem, wsem):
    wid = lax.axis_index("s") * _NC + lax.axis_index("c")
    pltpu.sync_copy(user_h.at[pl.ds(wid * _BPW, _BPW)], uidx)
    pltpu.sync_copy(item_h.at[pl.ds(wid * _BPW, _BPW)], iidx)
    writes = [None, None]
    for c in range(_NCHUNK):
      b = c % 2
      base = wid * _BPW + c * _CHUNK
      if writes[b] is not None:
        for w in writes[b]:
          w.wait()
      uc = uidx.at[pl.ds(c * _CHUNK, _CHUNK)]
      ic = iidx.at[pl.ds(c * _CHUNK, _CHUNK)]
      cc = pltpu.async_copy(mlpu_h.at[uc], bufc.at[b], gsem)
      cd = pltpu.async_copy(mlpi_h.at[ic], bufd.at[b], gsem)
      cc.wait()
      wc = pltpu.async_copy(bufc.at[b], omlpu_h.at[pl.ds(base, _CHUNK)], wsem)
      cd.wait()
      wd = pltpu.async_copy(bufd.at[b], omlpi_h.at[pl.ds(base, _CHUNK)], wsem)
      writes[b] = (wc, wd)
    for ws in writes:
      for w in ws:
        w.wait()

  return k(user, item, mlp_u, mlp_i)


def _sc_gather_mf(user, item, mfcat):
  @functools.partial(
      pl.kernel,
      out_type=[
          jax.ShapeDtypeStruct((BATCH, 2 * MF_DIM), jnp.float32),
          jax.ShapeDtypeStruct((BATCH, 2 * MF_DIM), jnp.float32),
      ],
      mesh=_MESH,
      scratch_types=[
          pltpu.VMEM((_BPW,), jnp.int32),
          pltpu.VMEM((_BPW,), jnp.int32),
          pltpu.VMEM((2, _CHUNK, 2 * MF_DIM), jnp.float32),
          pltpu.VMEM((2, _CHUNK, 2 * MF_DIM), jnp.float32),
          pltpu.SemaphoreType.DMA,
          pltpu.SemaphoreType.DMA,
      ],
  )
  def k(user_h, item_h, mfcat_h, omfu_h, omfi_h,
        uidx, iidx, bufa, bufb, gsem, wsem):
    wid = lax.axis_index("s") * _NC + lax.axis_index("c")
    pltpu.sync_copy(user_h.at[pl.ds(wid * _BPW, _BPW)], uidx)
    pltpu.sync_copy(item_h.at[pl.ds(wid * _BPW, _BPW)], iidx)
    writes = [None, None]
    for c in range(_NCHUNK):
      b = c % 2
      base = wid * _BPW + c * _CHUNK
      if writes[b] is not None:
        for w in writes[b]:
          w.wait()
      uc = uidx.at[pl.ds(c * _CHUNK, _CHUNK)]
      ic = iidx.at[pl.ds(c * _CHUNK, _CHUNK)]
      ca = pltpu.async_copy(mfcat_h.at[uc], bufa.at[b], gsem)
      cb = pltpu.async_copy(mfcat_h.at[ic], bufb.at[b], gsem)
      ca.wait()
      wa = pltpu.async_copy(bufa.at[b], omfu_h.at[pl.ds(base, _CHUNK)], wsem)
      cb.wait()
      wb = pltpu.async_copy(bufb.at[b], omfi_h.at[pl.ds(base, _CHUNK)], wsem)
      writes[b] = (wa, wb)
    for ws in writes:
      for w in ws:
        w.wait()

  return k(user, item, mfcat)


_BT = 2048


_TBC = 1024


def _tc_transpose_body(ut_ref, it_ref, eye_ref, out_ref):
  eye = eye_ref[...]
  left = jax.lax.dot_general(ut_ref[...], eye, (((0,), (0,)), ((), ())),
                             preferred_element_type=jnp.float32)
  right = jax.lax.dot_general(it_ref[...], eye, (((0,), (0,)), ((), ())),
                              preferred_element_type=jnp.float32)
  out_ref[...] = jnp.concatenate([left, right], axis=1)


def _tc_build_mfcat(mf_uT, mf_iT, n_rows):
  grid = (-(-n_rows // _TBC),)
  eye = jnp.eye(MF_DIM, dtype=jnp.float32)
  return pl.pallas_call(
      _tc_transpose_body,
      grid=grid,
      in_specs=[
          pl.BlockSpec((MF_DIM, _TBC), lambda i: (0, i)),
          pl.BlockSpec((MF_DIM, _TBC), lambda i: (0, i)),
          pl.BlockSpec((MF_DIM, MF_DIM), lambda i: (0, 0)),
      ],
      out_specs=pl.BlockSpec((_TBC, 2 * MF_DIM), lambda i: (i, 0)),
      out_shape=jax.ShapeDtypeStruct((n_rows, 2 * MF_DIM), jnp.float32),
      compiler_params=pltpu.CompilerParams(
          dimension_semantics=("arbitrary",)),
  )(mf_uT, mf_iT, eye)


def _tc_body(mfu, mfi, mlpu, mlpi, w1u, w1i, b1, w2, b2, wo, bo, out):
  x = jnp.dot(mlpu[...], w1u[...], preferred_element_type=jnp.float32)
  x = x + jnp.dot(mlpi[...], w1i[...], preferred_element_type=jnp.float32)
  h1 = jnp.maximum(x + b1[...], 0.0)
  h2 = jnp.maximum(
      jnp.dot(h1, w2[...], preferred_element_type=jnp.float32) + b2[...], 0.0)
  g = mfu[...][:, :MF_DIM] * mfi[...][:, MF_DIM:]
  p = jnp.concatenate([g, h2], axis=1)
  z = jnp.sum(p * wo[...], axis=1, keepdims=True) + bo[...]
  out[...] = jax.nn.sigmoid(z)


def _tc_mlp(mfu, mfi, mlpu, mlpi, W1, b1, W2, b2, W_out, b_out):
  w1t = W1.T
  w1u = w1t[:MLP_DIM]
  w1i = w1t[MLP_DIM:]
  w2t = W2.T
  b1r = b1.reshape(1, -1)
  b2r = b2.reshape(1, -1)
  wo = W_out.reshape(1, -1)
  bo = b_out.reshape(1, 1)

  grid = (BATCH // _BT,)
  bspec_row = lambda d: pl.BlockSpec((_BT, d), lambda i: (i, 0))
  bspec_full = lambda s: pl.BlockSpec(s, lambda i: (0, 0))
  return pl.pallas_call(
      _tc_body,
      grid=grid,
      in_specs=[
          bspec_row(2 * MF_DIM), bspec_row(2 * MF_DIM),
          bspec_row(MLP_DIM), bspec_row(MLP_DIM),
          bspec_full((MLP_DIM, 128)), bspec_full((MLP_DIM, 128)),
          bspec_full((1, 128)),
          bspec_full((128, 64)), bspec_full((1, 64)),
          bspec_full((1, 128)), bspec_full((1, 1)),
      ],
      out_specs=pl.BlockSpec((_BT, 1), lambda i: (i, 0)),
      out_shape=jax.ShapeDtypeStruct((BATCH, 1), jnp.float32),
      compiler_params=pltpu.CompilerParams(
          dimension_semantics=("arbitrary",)),
  )(mfu, mfi, mlpu, mlpi, w1u, w1i, b1r, w2t, b2r, wo, bo)


def kernel(user, item, mf_emb_user, mf_emb_item, mlp_emb_user, mlp_emb_item,
           W1, b1, W2, b2, W_out, b_out):
  user = user.astype(jnp.int32)
  item = item.astype(jnp.int32)
  mlpu, mlpi = _sc_gather_mlp(user, item, mlp_emb_user, mlp_emb_item)
  mfcat = _tc_build_mfcat(mf_emb_user.T, mf_emb_item.T,
                          mf_emb_user.shape[0])
  mfu, mfi = _sc_gather_mf(user, item, mfcat)
  return _tc_mlp(mfu, mfi, mlpu, mlpi, W1, b1, W2, b2, W_out, b_out)


# speedup vs baseline: 1.3591x; 1.0768x over previous
import functools

import jax
import jax.numpy as jnp
from jax import lax
from jax.experimental import pallas as pl
from jax.experimental.pallas import tpu as pltpu
from jax.experimental.pallas import tpu_sc as plsc

BATCH = 16384
MF_DIM = 64
MLP_DIM = 128

_NC = 2
_NS = 16
_NW = _NC * _NS
_BPW = BATCH // _NW
_CHUNK = 128
_NCHUNK = _BPW // _CHUNK


_MESH = plsc.VectorSubcoreMesh(
    core_axis_name="c", subcore_axis_name="s",
    num_cores=_NC, num_subcores=_NS)


def _sc_gather_mlp(user, item, mlp_u, mlp_i):
  @functools.partial(
      pl.kernel,
      out_type=[
          jax.ShapeDtypeStruct((BATCH, MLP_DIM), jnp.float32),
          jax.ShapeDtypeStruct((BATCH, MLP_DIM), jnp.float32),
      ],
      mesh=_MESH,
      scratch_types=[
          pltpu.VMEM((_BPW,), jnp.int32),
          pltpu.VMEM((_BPW,), jnp.int32),
          pltpu.VMEM((2, _CHUNK, MLP_DIM), jnp.float32),
          pltpu.VMEM((2, _CHUNK, MLP_DIM), jnp.float32),
          pltpu.SemaphoreType.DMA,
          pltpu.SemaphoreType.DMA,
      ],
  )
  def k(user_h, item_h, mlpu_h, mlpi_h, omlpu_h, omlpi_h,
        uidx, iidx, bufc, bufd, gsem, wsem):
    wid = lax.axis_index("s") * _NC + lax.axis_index("c")
    pltpu.sync_copy(user_h.at[pl.ds(wid * _BPW, _BPW)], uidx)
    pltpu.sync_copy(item_h.at[pl.ds(wid * _BPW, _BPW)], iidx)
    writes = [None, None]
    for c in range(_NCHUNK):
      b = c % 2
      base = wid * _BPW + c * _CHUNK
      if writes[b] is not None:
        for w in writes[b]:
          w.wait()
      uc = uidx.at[pl.ds(c * _CHUNK, _CHUNK)]
      ic = iidx.at[pl.ds(c * _CHUNK, _CHUNK)]
      cc = pltpu.async_copy(mlpu_h.at[uc], bufc.at[b], gsem)
      cd = pltpu.async_copy(mlpi_h.at[ic], bufd.at[b], gsem)
      cc.wait()
      wc = pltpu.async_copy(bufc.at[b], omlpu_h.at[pl.ds(base, _CHUNK)], wsem)
      cd.wait()
      wd = pltpu.async_copy(bufd.at[b], omlpi_h.at[pl.ds(base, _CHUNK)], wsem)
      writes[b] = (wc, wd)
    for ws in writes:
      for w in ws:
        w.wait()

  return k(user, item, mlp_u, mlp_i)


def _sc_gather_mf(user, item, mfcat):
  @functools.partial(
      pl.kernel,
      out_type=[
          jax.ShapeDtypeStruct((BATCH, 2 * MF_DIM), jnp.float32),
          jax.ShapeDtypeStruct((BATCH, 2 * MF_DIM), jnp.float32),
      ],
      mesh=_MESH,
      scratch_types=[
          pltpu.VMEM((_BPW,), jnp.int32),
          pltpu.VMEM((_BPW,), jnp.int32),
          pltpu.VMEM((2, _CHUNK, 2 * MF_DIM), jnp.float32),
          pltpu.VMEM((2, _CHUNK, 2 * MF_DIM), jnp.float32),
          pltpu.SemaphoreType.DMA,
          pltpu.SemaphoreType.DMA,
      ],
  )
  def k(user_h, item_h, mfcat_h, omfu_h, omfi_h,
        uidx, iidx, bufa, bufb, gsem, wsem):
    wid = lax.axis_index("s") * _NC + lax.axis_index("c")
    pltpu.sync_copy(user_h.at[pl.ds(wid * _BPW, _BPW)], uidx)
    pltpu.sync_copy(item_h.at[pl.ds(wid * _BPW, _BPW)], iidx)
    writes = [None, None]
    for c in range(_NCHUNK):
      b = c % 2
      base = wid * _BPW + c * _CHUNK
      if writes[b] is not None:
        for w in writes[b]:
          w.wait()
      uc = uidx.at[pl.ds(c * _CHUNK, _CHUNK)]
      ic = iidx.at[pl.ds(c * _CHUNK, _CHUNK)]
      ca = pltpu.async_copy(mfcat_h.at[uc], bufa.at[b], gsem)
      cb = pltpu.async_copy(mfcat_h.at[ic], bufb.at[b], gsem)
      ca.wait()
      wa = pltpu.async_copy(bufa.at[b], omfu_h.at[pl.ds(base, _CHUNK)], wsem)
      cb.wait()
      wb = pltpu.async_copy(bufb.at[b], omfi_h.at[pl.ds(base, _CHUNK)], wsem)
      writes[b] = (wa, wb)
    for ws in writes:
      for w in ws:
        w.wait()

  return k(user, item, mfcat)


_BT = 2048


_TBC = 1024


def _tc_transpose_body(ut_ref, it_ref, eye_ref, out_ref):
  x2 = jnp.concatenate([ut_ref[...], it_ref[...]], axis=0)
  out_ref[...] = jax.lax.dot_general(
      x2, eye_ref[...], (((0,), (0,)), ((), ())),
      preferred_element_type=jnp.float32)


def _tc_build_mfcat(mf_uT, mf_iT, n_rows):
  grid = (-(-n_rows // _TBC),)
  eye = jnp.eye(2 * MF_DIM, dtype=jnp.float32)
  return pl.pallas_call(
      _tc_transpose_body,
      grid=grid,
      in_specs=[
          pl.BlockSpec((MF_DIM, _TBC), lambda i: (0, i)),
          pl.BlockSpec((MF_DIM, _TBC), lambda i: (0, i)),
          pl.BlockSpec((2 * MF_DIM, 2 * MF_DIM), lambda i: (0, 0)),
      ],
      out_specs=pl.BlockSpec((_TBC, 2 * MF_DIM), lambda i: (i, 0)),
      out_shape=jax.ShapeDtypeStruct((n_rows, 2 * MF_DIM), jnp.float32),
      compiler_params=pltpu.CompilerParams(
          dimension_semantics=("arbitrary",),
          fuse_transposed_lhs_in_matmul=True),
  )(mf_uT, mf_iT, eye)


def _tc_body(mfu, mfi, mlpu, mlpi, w1u, w1i, b1, w2, b2, wo, bo, out):
  x = jnp.dot(mlpu[...], w1u[...], preferred_element_type=jnp.float32)
  x = x + jnp.dot(mlpi[...], w1i[...], preferred_element_type=jnp.float32)
  h1 = jnp.maximum(x + b1[...], 0.0)
  h2 = jnp.maximum(
      jnp.dot(h1, w2[...], preferred_element_type=jnp.float32) + b2[...], 0.0)
  g = mfu[...][:, :MF_DIM] * mfi[...][:, MF_DIM:]
  p = jnp.concatenate([g, h2], axis=1)
  z = jnp.sum(p * wo[...], axis=1, keepdims=True) + bo[...]
  out[...] = jax.nn.sigmoid(z)


def _tc_mlp(mfu, mfi, mlpu, mlpi, W1, b1, W2, b2, W_out, b_out):
  w1t = W1.T
  w1u = w1t[:MLP_DIM]
  w1i = w1t[MLP_DIM:]
  w2t = W2.T
  b1r = b1.reshape(1, -1)
  b2r = b2.reshape(1, -1)
  wo = W_out.reshape(1, -1)
  bo = b_out.reshape(1, 1)

  grid = (BATCH // _BT,)
  bspec_row = lambda d: pl.BlockSpec((_BT, d), lambda i: (i, 0))
  bspec_full = lambda s: pl.BlockSpec(s, lambda i: (0, 0))
  return pl.pallas_call(
      _tc_body,
      grid=grid,
      in_specs=[
          bspec_row(2 * MF_DIM), bspec_row(2 * MF_DIM),
          bspec_row(MLP_DIM), bspec_row(MLP_DIM),
          bspec_full((MLP_DIM, 128)), bspec_full((MLP_DIM, 128)),
          bspec_full((1, 128)),
          bspec_full((128, 64)), bspec_full((1, 64)),
          bspec_full((1, 128)), bspec_full((1, 1)),
      ],
      out_specs=pl.BlockSpec((_BT, 1), lambda i: (i, 0)),
      out_shape=jax.ShapeDtypeStruct((BATCH, 1), jnp.float32),
      compiler_params=pltpu.CompilerParams(
          dimension_semantics=("arbitrary",)),
  )(mfu, mfi, mlpu, mlpi, w1u, w1i, b1r, w2t, b2r, wo, bo)


def kernel(user, item, mf_emb_user, mf_emb_item, mlp_emb_user, mlp_emb_item,
           W1, b1, W2, b2, W_out, b_out):
  user = user.astype(jnp.int32)
  item = item.astype(jnp.int32)
  mlpu, mlpi = _sc_gather_mlp(user, item, mlp_emb_user, mlp_emb_item)
  mfcat = _tc_build_mfcat(mf_emb_user.T, mf_emb_item.T,
                          mf_emb_user.shape[0])
  mfu, mfi = _sc_gather_mf(user, item, mfcat)
  return _tc_mlp(mfu, mfi, mlpu, mlpi, W1, b1, W2, b2, W_out, b_out)


# speedup vs baseline: 1.8283x; 1.3452x over previous
import functools

import jax
import jax.numpy as jnp
from jax import lax
from jax.experimental import pallas as pl
from jax.experimental.pallas import tpu as pltpu
from jax.experimental.pallas import tpu_sc as plsc

BATCH = 16384
MF_DIM = 64
MLP_DIM = 128

_NC = 2
_NS = 16
_NW = _NC * _NS
_BPW = BATCH // _NW
_CHUNK = 128
_NCHUNK = _BPW // _CHUNK


_MESH = plsc.VectorSubcoreMesh(
    core_axis_name="c", subcore_axis_name="s",
    num_cores=_NC, num_subcores=_NS)


def _sc_gather_mlp(user, item, mlp_u, mlp_i):
  @functools.partial(
      pl.kernel,
      out_type=[
          jax.ShapeDtypeStruct((BATCH, MLP_DIM), jnp.float32),
          jax.ShapeDtypeStruct((BATCH, MLP_DIM), jnp.float32),
      ],
      mesh=_MESH,
      compiler_params=pltpu.CompilerParams(has_side_effects=True),
      scratch_types=[
          pltpu.VMEM((_BPW,), jnp.int32),
          pltpu.VMEM((_BPW,), jnp.int32),
          pltpu.VMEM((2, _CHUNK, MLP_DIM), jnp.float32),
          pltpu.VMEM((2, _CHUNK, MLP_DIM), jnp.float32),
          pltpu.SemaphoreType.DMA,
          pltpu.SemaphoreType.DMA,
      ],
  )
  def k(user_h, item_h, mlpu_h, mlpi_h, omlpu_h, omlpi_h,
        uidx, iidx, bufc, bufd, gsem, wsem):
    wid = lax.axis_index("s") * _NC + lax.axis_index("c")
    pltpu.sync_copy(user_h.at[pl.ds(wid * _BPW, _BPW)], uidx)
    pltpu.sync_copy(item_h.at[pl.ds(wid * _BPW, _BPW)], iidx)
    writes = [None, None]
    for c in range(_NCHUNK):
      b = c % 2
      base = wid * _BPW + c * _CHUNK
      if writes[b] is not None:
        for w in writes[b]:
          w.wait()
      uc = uidx.at[pl.ds(c * _CHUNK, _CHUNK)]
      ic = iidx.at[pl.ds(c * _CHUNK, _CHUNK)]
      cc = pltpu.async_copy(mlpu_h.at[uc], bufc.at[b], gsem)
      cd = pltpu.async_copy(mlpi_h.at[ic], bufd.at[b], gsem)
      cc.wait()
      wc = pltpu.async_copy(bufc.at[b], omlpu_h.at[pl.ds(base, _CHUNK)], wsem)
      cd.wait()
      wd = pltpu.async_copy(bufd.at[b], omlpi_h.at[pl.ds(base, _CHUNK)], wsem)
      writes[b] = (wc, wd)
    for ws in writes:
      for w in ws:
        w.wait()

  return k(user, item, mlp_u, mlp_i)


def _sc_gather_mf(user, item, mfcat):
  @functools.partial(
      pl.kernel,
      out_type=[
          jax.ShapeDtypeStruct((BATCH, 2 * MF_DIM), jnp.float32),
          jax.ShapeDtypeStruct((BATCH, 2 * MF_DIM), jnp.float32),
      ],
      mesh=_MESH,
      scratch_types=[
          pltpu.VMEM((_BPW,), jnp.int32),
          pltpu.VMEM((_BPW,), jnp.int32),
          pltpu.VMEM((2, _CHUNK, 2 * MF_DIM), jnp.float32),
          pltpu.VMEM((2, _CHUNK, 2 * MF_DIM), jnp.float32),
          pltpu.SemaphoreType.DMA,
          pltpu.SemaphoreType.DMA,
      ],
  )
  def k(user_h, item_h, mfcat_h, omfu_h, omfi_h,
        uidx, iidx, bufa, bufb, gsem, wsem):
    wid = lax.axis_index("s") * _NC + lax.axis_index("c")
    pltpu.sync_copy(user_h.at[pl.ds(wid * _BPW, _BPW)], uidx)
    pltpu.sync_copy(item_h.at[pl.ds(wid * _BPW, _BPW)], iidx)
    writes = [None, None]
    for c in range(_NCHUNK):
      b = c % 2
      base = wid * _BPW + c * _CHUNK
      if writes[b] is not None:
        for w in writes[b]:
          w.wait()
      uc = uidx.at[pl.ds(c * _CHUNK, _CHUNK)]
      ic = iidx.at[pl.ds(c * _CHUNK, _CHUNK)]
      ca = pltpu.async_copy(mfcat_h.at[uc], bufa.at[b], gsem)
      cb = pltpu.async_copy(mfcat_h.at[ic], bufb.at[b], gsem)
      ca.wait()
      wa = pltpu.async_copy(bufa.at[b], omfu_h.at[pl.ds(base, _CHUNK)], wsem)
      cb.wait()
      wb = pltpu.async_copy(bufb.at[b], omfi_h.at[pl.ds(base, _CHUNK)], wsem)
      writes[b] = (wa, wb)
    for ws in writes:
      for w in ws:
        w.wait()

  return k(user, item, mfcat)


_BT = 2048


_TBC = 4096


def _tc_transpose_body(ut_ref, it_ref, eye_ref, out_ref):
  x2 = jnp.concatenate([ut_ref[...], it_ref[...]], axis=0)
  out_ref[...] = jax.lax.dot_general(
      x2, eye_ref[...], (((0,), (0,)), ((), ())),
      preferred_element_type=jnp.float32)


def _tc_build_mfcat(mf_uT, mf_iT, n_rows):
  grid = (-(-n_rows // _TBC),)
  eye = jnp.eye(2 * MF_DIM, dtype=jnp.float32)
  return pl.pallas_call(
      _tc_transpose_body,
      grid=grid,
      in_specs=[
          pl.BlockSpec((MF_DIM, _TBC), lambda i: (0, i)),
          pl.BlockSpec((MF_DIM, _TBC), lambda i: (0, i)),
          pl.BlockSpec((2 * MF_DIM, 2 * MF_DIM), lambda i: (0, 0)),
      ],
      out_specs=pl.BlockSpec((_TBC, 2 * MF_DIM), lambda i: (i, 0)),
      out_shape=jax.ShapeDtypeStruct((n_rows, 2 * MF_DIM), jnp.float32),
      compiler_params=pltpu.CompilerParams(
          dimension_semantics=("arbitrary",),
          fuse_transposed_lhs_in_matmul=True),
  )(mf_uT, mf_iT, eye)


def _tc_body(mfu, mfi, mlpu, mlpi, w1u, w1i, b1, w2, b2, wo, bo, out):
  x = jnp.dot(mlpu[...], w1u[...], preferred_element_type=jnp.float32)
  x = x + jnp.dot(mlpi[...], w1i[...], preferred_element_type=jnp.float32)
  h1 = jnp.maximum(x + b1[...], 0.0)
  h2 = jnp.maximum(
      jnp.dot(h1, w2[...], preferred_element_type=jnp.float32) + b2[...], 0.0)
  g = mfu[...][:, :MF_DIM] * mfi[...][:, MF_DIM:]
  p = jnp.concatenate([g, h2], axis=1)
  z = jnp.sum(p * wo[...], axis=1, keepdims=True) + bo[...]
  out[...] = jax.nn.sigmoid(z)


def _tc_mlp(mfu, mfi, mlpu, mlpi, W1, b1, W2, b2, W_out, b_out):
  w1t = W1.T
  w1u = w1t[:MLP_DIM]
  w1i = w1t[MLP_DIM:]
  w2t = W2.T
  b1r = b1.reshape(1, -1)
  b2r = b2.reshape(1, -1)
  wo = W_out.reshape(1, -1)
  bo = b_out.reshape(1, 1)

  grid = (BATCH // _BT,)
  bspec_row = lambda d: pl.BlockSpec((_BT, d), lambda i: (i, 0))
  bspec_full = lambda s: pl.BlockSpec(s, lambda i: (0, 0))
  return pl.pallas_call(
      _tc_body,
      grid=grid,
      in_specs=[
          bspec_row(2 * MF_DIM), bspec_row(2 * MF_DIM),
          bspec_row(MLP_DIM), bspec_row(MLP_DIM),
          bspec_full((MLP_DIM, 128)), bspec_full((MLP_DIM, 128)),
          bspec_full((1, 128)),
          bspec_full((128, 64)), bspec_full((1, 64)),
          bspec_full((1, 128)), bspec_full((1, 1)),
      ],
      out_specs=pl.BlockSpec((_BT, 1), lambda i: (i, 0)),
      out_shape=jax.ShapeDtypeStruct((BATCH, 1), jnp.float32),
      compiler_params=pltpu.CompilerParams(
          dimension_semantics=("arbitrary",)),
  )(mfu, mfi, mlpu, mlpi, w1u, w1i, b1r, w2t, b2r, wo, bo)


def kernel(user, item, mf_emb_user, mf_emb_item, mlp_emb_user, mlp_emb_item,
           W1, b1, W2, b2, W_out, b_out):
  user = user.astype(jnp.int32)
  item = item.astype(jnp.int32)
  mlpu, mlpi = _sc_gather_mlp(user, item, mlp_emb_user, mlp_emb_item)
  mfcat = _tc_build_mfcat(mf_emb_user.T, mf_emb_item.T,
                          mf_emb_user.shape[0])
  mfu, mfi = _sc_gather_mf(user, item, mfcat)
  return _tc_mlp(mfu, mfi, mlpu, mlpi, W1, b1, W2, b2, W_out, b_out)


# speedup vs baseline: 1.8355x; 1.0040x over previous
import functools

import jax
import jax.numpy as jnp
from jax import lax
from jax.experimental import pallas as pl
from jax.experimental.pallas import tpu as pltpu
from jax.experimental.pallas import tpu_sc as plsc

BATCH = 16384
MF_DIM = 64
MLP_DIM = 128

_NC = 2
_NS = 16
_NW = _NC * _NS
_BPW = BATCH // _NW
_CHUNK = 128
_NCHUNK = _BPW // _CHUNK


_MESH = plsc.VectorSubcoreMesh(
    core_axis_name="c", subcore_axis_name="s",
    num_cores=_NC, num_subcores=_NS)


def _sc_gather_mlp(user, item, mlp_u, mlp_i):
  @functools.partial(
      pl.kernel,
      out_type=[
          jax.ShapeDtypeStruct((BATCH, MLP_DIM), jnp.float32),
          jax.ShapeDtypeStruct((BATCH, MLP_DIM), jnp.float32),
      ],
      mesh=_MESH,
      compiler_params=pltpu.CompilerParams(skip_device_barrier=True),
      scratch_types=[
          pltpu.VMEM((_BPW,), jnp.int32),
          pltpu.VMEM((_BPW,), jnp.int32),
          pltpu.VMEM((2, _CHUNK, MLP_DIM), jnp.float32),
          pltpu.VMEM((2, _CHUNK, MLP_DIM), jnp.float32),
          pltpu.SemaphoreType.DMA,
          pltpu.SemaphoreType.DMA,
      ],
  )
  def k(user_h, item_h, mlpu_h, mlpi_h, omlpu_h, omlpi_h,
        uidx, iidx, bufc, bufd, gsem, wsem):
    wid = lax.axis_index("s") * _NC + lax.axis_index("c")
    pltpu.sync_copy(user_h.at[pl.ds(wid * _BPW, _BPW)], uidx)
    pltpu.sync_copy(item_h.at[pl.ds(wid * _BPW, _BPW)], iidx)
    writes = [None, None]
    for c in range(_NCHUNK):
      b = c % 2
      base = wid * _BPW + c * _CHUNK
      if writes[b] is not None:
        for w in writes[b]:
          w.wait()
      uc = uidx.at[pl.ds(c * _CHUNK, _CHUNK)]
      ic = iidx.at[pl.ds(c * _CHUNK, _CHUNK)]
      cc = pltpu.async_copy(mlpu_h.at[uc], bufc.at[b], gsem)
      cd = pltpu.async_copy(mlpi_h.at[ic], bufd.at[b], gsem)
      cc.wait()
      wc = pltpu.async_copy(bufc.at[b], omlpu_h.at[pl.ds(base, _CHUNK)], wsem)
      cd.wait()
      wd = pltpu.async_copy(bufd.at[b], omlpi_h.at[pl.ds(base, _CHUNK)], wsem)
      writes[b] = (wc, wd)
    for ws in writes:
      for w in ws:
        w.wait()

  return k(user, item, mlp_u, mlp_i)


def _sc_gather_mf(user, item, mfcat):
  @functools.partial(
      pl.kernel,
      out_type=[
          jax.ShapeDtypeStruct((BATCH, 2 * MF_DIM), jnp.float32),
          jax.ShapeDtypeStruct((BATCH, 2 * MF_DIM), jnp.float32),
      ],
      mesh=_MESH,
      compiler_params=pltpu.CompilerParams(skip_device_barrier=True),
      scratch_types=[
          pltpu.VMEM((_BPW,), jnp.int32),
          pltpu.VMEM((_BPW,), jnp.int32),
          pltpu.VMEM((2, _CHUNK, 2 * MF_DIM), jnp.float32),
          pltpu.VMEM((2, _CHUNK, 2 * MF_DIM), jnp.float32),
          pltpu.SemaphoreType.DMA,
          pltpu.SemaphoreType.DMA,
      ],
  )
  def k(user_h, item_h, mfcat_h, omfu_h, omfi_h,
        uidx, iidx, bufa, bufb, gsem, wsem):
    wid = lax.axis_index("s") * _NC + lax.axis_index("c")
    pltpu.sync_copy(user_h.at[pl.ds(wid * _BPW, _BPW)], uidx)
    pltpu.sync_copy(item_h.at[pl.ds(wid * _BPW, _BPW)], iidx)
    writes = [None, None]
    for c in range(_NCHUNK):
      b = c % 2
      base = wid * _BPW + c * _CHUNK
      if writes[b] is not None:
        for w in writes[b]:
          w.wait()
      uc = uidx.at[pl.ds(c * _CHUNK, _CHUNK)]
      ic = iidx.at[pl.ds(c * _CHUNK, _CHUNK)]
      ca = pltpu.async_copy(mfcat_h.at[uc], bufa.at[b], gsem)
      cb = pltpu.async_copy(mfcat_h.at[ic], bufb.at[b], gsem)
      ca.wait()
      wa = pltpu.async_copy(bufa.at[b], omfu_h.at[pl.ds(base, _CHUNK)], wsem)
      cb.wait()
      wb = pltpu.async_copy(bufb.at[b], omfi_h.at[pl.ds(base, _CHUNK)], wsem)
      writes[b] = (wa, wb)
    for ws in writes:
      for w in ws:
        w.wait()

  return k(user, item, mfcat)


_BT = 2048


_TBC = 4096


def _tc_transpose_body(ut_ref, it_ref, eye_ref, out_ref):
  x2 = jnp.concatenate([ut_ref[...], it_ref[...]], axis=0)
  out_ref[...] = jax.lax.dot_general(
      x2, eye_ref[...], (((0,), (0,)), ((), ())),
      preferred_element_type=jnp.float32)


def _tc_build_mfcat(mf_uT, mf_iT, n_rows):
  grid = (-(-n_rows // _TBC),)
  eye = jnp.eye(2 * MF_DIM, dtype=jnp.float32)
  return pl.pallas_call(
      _tc_transpose_body,
      grid=grid,
      in_specs=[
          pl.BlockSpec((MF_DIM, _TBC), lambda i: (0, i)),
          pl.BlockSpec((MF_DIM, _TBC), lambda i: (0, i)),
          pl.BlockSpec((2 * MF_DIM, 2 * MF_DIM), lambda i: (0, 0)),
      ],
      out_specs=pl.BlockSpec((_TBC, 2 * MF_DIM), lambda i: (i, 0)),
      out_shape=jax.ShapeDtypeStruct((n_rows, 2 * MF_DIM), jnp.float32),
      compiler_params=pltpu.CompilerParams(
          dimension_semantics=("arbitrary",),
          fuse_transposed_lhs_in_matmul=True,
          skip_device_barrier=True),
  )(mf_uT, mf_iT, eye)


def _tc_body(mfu, mfi, mlpu, mlpi, w1u, w1i, b1, w2, b2, wo, bo, out):
  x = jnp.dot(mlpu[...], w1u[...], preferred_element_type=jnp.float32)
  x = x + jnp.dot(mlpi[...], w1i[...], preferred_element_type=jnp.float32)
  h1 = jnp.maximum(x + b1[...], 0.0)
  h2 = jnp.maximum(
      jnp.dot(h1, w2[...], preferred_element_type=jnp.float32) + b2[...], 0.0)
  g = mfu[...][:, :MF_DIM] * mfi[...][:, MF_DIM:]
  p = jnp.concatenate([g, h2], axis=1)
  z = jnp.sum(p * wo[...], axis=1, keepdims=True) + bo[...]
  out[...] = jax.nn.sigmoid(z)


def _tc_mlp(mfu, mfi, mlpu, mlpi, W1, b1, W2, b2, W_out, b_out):
  w1t = W1.T
  w1u = w1t[:MLP_DIM]
  w1i = w1t[MLP_DIM:]
  w2t = W2.T
  b1r = b1.reshape(1, -1)
  b2r = b2.reshape(1, -1)
  wo = W_out.reshape(1, -1)
  bo = b_out.reshape(1, 1)

  grid = (BATCH // _BT,)
  bspec_row = lambda d: pl.BlockSpec((_BT, d), lambda i: (i, 0))
  bspec_full = lambda s: pl.BlockSpec(s, lambda i: (0, 0))
  return pl.pallas_call(
      _tc_body,
      grid=grid,
      in_specs=[
          bspec_row(2 * MF_DIM), bspec_row(2 * MF_DIM),
          bspec_row(MLP_DIM), bspec_row(MLP_DIM),
          bspec_full((MLP_DIM, 128)), bspec_full((MLP_DIM, 128)),
          bspec_full((1, 128)),
          bspec_full((128, 64)), bspec_full((1, 64)),
          bspec_full((1, 128)), bspec_full((1, 1)),
      ],
      out_specs=pl.BlockSpec((_BT, 1), lambda i: (i, 0)),
      out_shape=jax.ShapeDtypeStruct((BATCH, 1), jnp.float32),
      compiler_params=pltpu.CompilerParams(
          dimension_semantics=("arbitrary",),
          skip_device_barrier=True),
  )(mfu, mfi, mlpu, mlpi, w1u, w1i, b1r, w2t, b2r, wo, bo)


def kernel(user, item, mf_emb_user, mf_emb_item, mlp_emb_user, mlp_emb_item,
           W1, b1, W2, b2, W_out, b_out):
  user = user.astype(jnp.int32)
  item = item.astype(jnp.int32)
  mlpu, mlpi = _sc_gather_mlp(user, item, mlp_emb_user, mlp_emb_item)
  mfcat = _tc_build_mfcat(mf_emb_user.T, mf_emb_item.T,
                          mf_emb_user.shape[0])
  mfu, mfi = _sc_gather_mf(user, item, mfcat)
  return _tc_mlp(mfu, mfi, mlpu, mlpi, W1, b1, W2, b2, W_out, b_out)


# speedup vs baseline: 1.9207x; 1.0464x over previous
import functools

import jax
import jax.numpy as jnp
from jax import lax
from jax.experimental import pallas as pl
from jax.experimental.pallas import tpu as pltpu
from jax.experimental.pallas import tpu_sc as plsc

BATCH = 16384
MF_DIM = 64
MLP_DIM = 128

_NC = 2
_NS = 16
_NW = _NC * _NS
_BPW = BATCH // _NW
_CHUNK = 128
_NCHUNK = _BPW // _CHUNK


_MESH = plsc.VectorSubcoreMesh(
    core_axis_name="c", subcore_axis_name="s",
    num_cores=_NC, num_subcores=_NS)


_GCH = 64
_NGCH = _BPW // _GCH


def _sc_gather_all(user, item, mlp_u, mlp_i, mfcat):
  @functools.partial(
      pl.kernel,
      out_type=[
          jax.ShapeDtypeStruct((BATCH, MLP_DIM), jnp.float32),
          jax.ShapeDtypeStruct((BATCH, MLP_DIM), jnp.float32),
          jax.ShapeDtypeStruct((BATCH, 2 * MF_DIM), jnp.float32),
          jax.ShapeDtypeStruct((BATCH, 2 * MF_DIM), jnp.float32),
      ],
      mesh=_MESH,
      compiler_params=pltpu.CompilerParams(skip_device_barrier=True),
      scratch_types=[
          pltpu.VMEM((_BPW,), jnp.int32),
          pltpu.VMEM((_BPW,), jnp.int32),
          pltpu.VMEM((2, _GCH, MLP_DIM), jnp.float32),
          pltpu.VMEM((2, _GCH, MLP_DIM), jnp.float32),
          pltpu.VMEM((2, _GCH, 2 * MF_DIM), jnp.float32),
          pltpu.VMEM((2, _GCH, 2 * MF_DIM), jnp.float32),
          pltpu.SemaphoreType.DMA,
          pltpu.SemaphoreType.DMA,
      ],
  )
  def k(user_h, item_h, mlpu_h, mlpi_h, mfcat_h,
        omlpu_h, omlpi_h, omfu_h, omfi_h,
        uidx, iidx, bufc, bufd, bufa, bufb, gsem, wsem):
    wid = lax.axis_index("s") * _NC + lax.axis_index("c")
    pltpu.sync_copy(user_h.at[pl.ds(wid * _BPW, _BPW)], uidx)
    pltpu.sync_copy(item_h.at[pl.ds(wid * _BPW, _BPW)], iidx)
    writes = [None, None]
    for c in range(_NGCH):
      b = c % 2
      base = wid * _BPW + c * _GCH
      if writes[b] is not None:
        for w in writes[b]:
          w.wait()
      uc = uidx.at[pl.ds(c * _GCH, _GCH)]
      ic = iidx.at[pl.ds(c * _GCH, _GCH)]
      cc = pltpu.async_copy(mlpu_h.at[uc], bufc.at[b], gsem)
      cd = pltpu.async_copy(mlpi_h.at[ic], bufd.at[b], gsem)
      ca = pltpu.async_copy(mfcat_h.at[uc], bufa.at[b], gsem)
      cb = pltpu.async_copy(mfcat_h.at[ic], bufb.at[b], gsem)
      cc.wait()
      wc = pltpu.async_copy(bufc.at[b], omlpu_h.at[pl.ds(base, _GCH)], wsem)
      cd.wait()
      wd = pltpu.async_copy(bufd.at[b], omlpi_h.at[pl.ds(base, _GCH)], wsem)
      ca.wait()
      wa = pltpu.async_copy(bufa.at[b], omfu_h.at[pl.ds(base, _GCH)], wsem)
      cb.wait()
      wb = pltpu.async_copy(bufb.at[b], omfi_h.at[pl.ds(base, _GCH)], wsem)
      writes[b] = (wc, wd, wa, wb)
    for ws in writes:
      for w in ws:
        w.wait()

  return k(user, item, mlp_u, mlp_i, mfcat)


_BT = 2048


_TBC = 4096


def _tc_transpose_body(ut_ref, it_ref, eye_ref, out_ref):
  x2 = jnp.concatenate([ut_ref[...], it_ref[...]], axis=0)
  out_ref[...] = jax.lax.dot_general(
      x2, eye_ref[...], (((0,), (0,)), ((), ())),
      preferred_element_type=jnp.float32)


def _tc_build_mfcat(mf_uT, mf_iT, n_rows):
  grid = (-(-n_rows // _TBC),)
  eye = jnp.eye(2 * MF_DIM, dtype=jnp.float32)
  return pl.pallas_call(
      _tc_transpose_body,
      grid=grid,
      in_specs=[
          pl.BlockSpec((MF_DIM, _TBC), lambda i: (0, i)),
          pl.BlockSpec((MF_DIM, _TBC), lambda i: (0, i)),
          pl.BlockSpec((2 * MF_DIM, 2 * MF_DIM), lambda i: (0, 0)),
      ],
      out_specs=pl.BlockSpec((_TBC, 2 * MF_DIM), lambda i: (i, 0)),
      out_shape=jax.ShapeDtypeStruct((n_rows, 2 * MF_DIM), jnp.float32),
      compiler_params=pltpu.CompilerParams(
          dimension_semantics=("arbitrary",),
          fuse_transposed_lhs_in_matmul=True,
          skip_device_barrier=True),
  )(mf_uT, mf_iT, eye)


def _tc_body(mfu, mfi, mlpu, mlpi, w1u, w1i, b1, w2, b2, wo, bo, out):
  x = jnp.dot(mlpu[...], w1u[...], preferred_element_type=jnp.float32)
  x = x + jnp.dot(mlpi[...], w1i[...], preferred_element_type=jnp.float32)
  h1 = jnp.maximum(x + b1[...], 0.0)
  h2 = jnp.maximum(
      jnp.dot(h1, w2[...], preferred_element_type=jnp.float32) + b2[...], 0.0)
  g = mfu[...][:, :MF_DIM] * mfi[...][:, MF_DIM:]
  p = jnp.concatenate([g, h2], axis=1)
  z = jnp.sum(p * wo[...], axis=1, keepdims=True) + bo[...]
  out[...] = jax.nn.sigmoid(z)


def _tc_mlp(mfu, mfi, mlpu, mlpi, W1, b1, W2, b2, W_out, b_out):
  w1t = W1.T
  w1u = w1t[:MLP_DIM]
  w1i = w1t[MLP_DIM:]
  w2t = W2.T
  b1r = b1.reshape(1, -1)
  b2r = b2.reshape(1, -1)
  wo = W_out.reshape(1, -1)
  bo = b_out.reshape(1, 1)

  grid = (BATCH // _BT,)
  bspec_row = lambda d: pl.BlockSpec((_BT, d), lambda i: (i, 0))
  bspec_full = lambda s: pl.BlockSpec(s, lambda i: (0, 0))
  return pl.pallas_call(
      _tc_body,
      grid=grid,
      in_specs=[
          bspec_row(2 * MF_DIM), bspec_row(2 * MF_DIM),
          bspec_row(MLP_DIM), bspec_row(MLP_DIM),
          bspec_full((MLP_DIM, 128)), bspec_full((MLP_DIM, 128)),
          bspec_full((1, 128)),
          bspec_full((128, 64)), bspec_full((1, 64)),
          bspec_full((1, 128)), bspec_full((1, 1)),
      ],
      out_specs=pl.BlockSpec((_BT, 1), lambda i: (i, 0)),
      out_shape=jax.ShapeDtypeStruct((BATCH, 1), jnp.float32),
      compiler_params=pltpu.CompilerParams(
          dimension_semantics=("arbitrary",),
          skip_device_barrier=True),
  )(mfu, mfi, mlpu, mlpi, w1u, w1i, b1r, w2t, b2r, wo, bo)


def kernel(user, item, mf_emb_user, mf_emb_item, mlp_emb_user, mlp_emb_item,
           W1, b1, W2, b2, W_out, b_out):
  user = user.astype(jnp.int32)
  item = item.astype(jnp.int32)
  mfcat = _tc_build_mfcat(mf_emb_user.T, mf_emb_item.T,
                          mf_emb_user.shape[0])
  mlpu, mlpi, mfu, mfi = _sc_gather_all(
      user, item, mlp_emb_user, mlp_emb_item, mfcat)
  return _tc_mlp(mfu, mfi, mlpu, mlpi, W1, b1, W2, b2, W_out, b_out)


# speedup vs baseline: 2.3598x; 1.2286x over previous
import functools

import jax
import jax.numpy as jnp
from jax import lax
from jax.experimental import pallas as pl
from jax.experimental.pallas import tpu as pltpu
from jax.experimental.pallas import tpu_sc as plsc

BATCH = 16384
MF_DIM = 64
MLP_DIM = 128

_NC = 2
_NS = 16
_NW = _NC * _NS
_BPW = BATCH // _NW
_CHUNK = 128
_NCHUNK = _BPW // _CHUNK


_MESH = plsc.VectorSubcoreMesh(
    core_axis_name="c", subcore_axis_name="s",
    num_cores=_NC, num_subcores=_NS)


_GCH = 64
_NGCH = _BPW // _GCH


def _sc_gather_all(user, item, mlp_u, mlp_i, mfcat):
  @functools.partial(
      pl.kernel,
      out_type=[
          jax.ShapeDtypeStruct((BATCH, MLP_DIM), jnp.float32),
          jax.ShapeDtypeStruct((BATCH, MLP_DIM), jnp.float32),
          jax.ShapeDtypeStruct((BATCH, 2 * MF_DIM), jnp.float32),
          jax.ShapeDtypeStruct((BATCH, 2 * MF_DIM), jnp.float32),
      ],
      mesh=_MESH,
      compiler_params=pltpu.CompilerParams(skip_device_barrier=True),
      scratch_types=[
          pltpu.VMEM((_BPW,), jnp.int32),
          pltpu.VMEM((_BPW,), jnp.int32),
          pltpu.VMEM((2, _GCH, MLP_DIM), jnp.float32),
          pltpu.VMEM((2, _GCH, MLP_DIM), jnp.float32),
          pltpu.VMEM((2, _GCH, 2 * MF_DIM), jnp.float32),
          pltpu.VMEM((2, _GCH, 2 * MF_DIM), jnp.float32),
          pltpu.SemaphoreType.DMA,
          pltpu.SemaphoreType.DMA,
      ],
  )
  def k(user_h, item_h, mlpu_h, mlpi_h, mfcat_h,
        omlpu_h, omlpi_h, omfu_h, omfi_h,
        uidx, iidx, bufc, bufd, bufa, bufb, gsem, wsem):
    wid = lax.axis_index("s") * _NC + lax.axis_index("c")
    pltpu.sync_copy(user_h.at[pl.ds(wid * _BPW, _BPW)], uidx)
    pltpu.sync_copy(item_h.at[pl.ds(wid * _BPW, _BPW)], iidx)
    writes = [None, None]
    for c in range(_NGCH):
      b = c % 2
      base = wid * _BPW + c * _GCH
      if writes[b] is not None:
        for w in writes[b]:
          w.wait()
      uc = uidx.at[pl.ds(c * _GCH, _GCH)]
      ic = iidx.at[pl.ds(c * _GCH, _GCH)]
      cc = pltpu.async_copy(mlpu_h.at[uc], bufc.at[b], gsem)
      cd = pltpu.async_copy(mlpi_h.at[ic], bufd.at[b], gsem)
      ca = pltpu.async_copy(mfcat_h.at[uc], bufa.at[b], gsem)
      cb = pltpu.async_copy(mfcat_h.at[ic], bufb.at[b], gsem)
      cc.wait()
      wc = pltpu.async_copy(bufc.at[b], omlpu_h.at[pl.ds(base, _GCH)], wsem)
      cd.wait()
      wd = pltpu.async_copy(bufd.at[b], omlpi_h.at[pl.ds(base, _GCH)], wsem)
      ca.wait()
      wa = pltpu.async_copy(bufa.at[b], omfu_h.at[pl.ds(base, _GCH)], wsem)
      cb.wait()
      wb = pltpu.async_copy(bufb.at[b], omfi_h.at[pl.ds(base, _GCH)], wsem)
      writes[b] = (wc, wd, wa, wb)
    for ws in writes:
      for w in ws:
        w.wait()

  return k(user, item, mlp_u, mlp_i, mfcat)


_BT = 4096


_TBC = 8192


def _tc_transpose_body(ut_ref, it_ref, eye_ref, out_ref):
  x2 = jnp.concatenate([ut_ref[...], it_ref[...]], axis=0)
  out_ref[...] = jax.lax.dot_general(
      x2, eye_ref[...], (((0,), (0,)), ((), ())),
      preferred_element_type=jnp.float32)


def _tc_build_mfcat(mf_uT, mf_iT, n_rows):
  grid = (-(-n_rows // _TBC),)
  eye = jnp.eye(2 * MF_DIM, dtype=jnp.float32)
  return pl.pallas_call(
      _tc_transpose_body,
      grid=grid,
      in_specs=[
          pl.BlockSpec((MF_DIM, _TBC), lambda i: (0, i)),
          pl.BlockSpec((MF_DIM, _TBC), lambda i: (0, i)),
          pl.BlockSpec((2 * MF_DIM, 2 * MF_DIM), lambda i: (0, 0)),
      ],
      out_specs=pl.BlockSpec((_TBC, 2 * MF_DIM), lambda i: (i, 0)),
      out_shape=jax.ShapeDtypeStruct((n_rows, 2 * MF_DIM), jnp.float32),
      compiler_params=pltpu.CompilerParams(
          dimension_semantics=("arbitrary",),
          fuse_transposed_lhs_in_matmul=True,
          skip_device_barrier=True),
  )(mf_uT, mf_iT, eye)


def _tc_body(mfu, mfi, mlpu, mlpi, w1u, w1i, b1, w2, b2, wo, bo, out):
  x = jnp.dot(mlpu[...], w1u[...], preferred_element_type=jnp.float32)
  x = x + jnp.dot(mlpi[...], w1i[...], preferred_element_type=jnp.float32)
  h1 = jnp.maximum(x + b1[...], 0.0)
  h2 = jnp.maximum(
      jnp.dot(h1, w2[...], preferred_element_type=jnp.float32) + b2[...], 0.0)
  g = mfu[...][:, :MF_DIM] * mfi[...][:, MF_DIM:]
  p = jnp.concatenate([g, h2], axis=1)
  z = jax.lax.dot_general(wo[...], p, (((1,), (1,)), ((), ())),
                          preferred_element_type=jnp.float32)
  out[...] = jax.nn.sigmoid(z + bo[...])


def _tc_mlp(mfu, mfi, mlpu, mlpi, W1, b1, W2, b2, W_out, b_out):
  w1t = W1.T
  w1u = w1t[:MLP_DIM]
  w1i = w1t[MLP_DIM:]
  w2t = W2.T
  b1r = b1.reshape(1, -1)
  b2r = b2.reshape(1, -1)
  wo = W_out.reshape(1, -1)
  bo = b_out.reshape(1, 1)

  grid = (BATCH // _BT,)
  bspec_row = lambda d: pl.BlockSpec((_BT, d), lambda i: (i, 0))
  bspec_full = lambda s: pl.BlockSpec(s, lambda i: (0, 0))
  return pl.pallas_call(
      _tc_body,
      grid=grid,
      in_specs=[
          bspec_row(2 * MF_DIM), bspec_row(2 * MF_DIM),
          bspec_row(MLP_DIM), bspec_row(MLP_DIM),
          bspec_full((MLP_DIM, 128)), bspec_full((MLP_DIM, 128)),
          bspec_full((1, 128)),
          bspec_full((128, 64)), bspec_full((1, 64)),
          bspec_full((1, 128)), bspec_full((1, 1)),
      ],
      out_specs=pl.BlockSpec((1, _BT), lambda i: (0, i)),
      out_shape=jax.ShapeDtypeStruct((1, BATCH), jnp.float32),
      compiler_params=pltpu.CompilerParams(
          dimension_semantics=("arbitrary",),
          skip_device_barrier=True),
  )(mfu, mfi, mlpu, mlpi, w1u, w1i, b1r, w2t, b2r, wo, bo)


def kernel(user, item, mf_emb_user, mf_emb_item, mlp_emb_user, mlp_emb_item,
           W1, b1, W2, b2, W_out, b_out):
  user = user.astype(jnp.int32)
  item = item.astype(jnp.int32)
  mfcat = _tc_build_mfcat(mf_emb_user.T, mf_emb_item.T,
                          mf_emb_user.shape[0])
  mlpu, mlpi, mfu, mfi = _sc_gather_all(
      user, item, mlp_emb_user, mlp_emb_item, mfcat)
  return _tc_mlp(mfu, mfi, mlpu, mlpi, W1, b1, W2, b2, W_out, b_out).T
